# baseline (speedup 1.0000x reference)
import functools

import jax
import jax.numpy as jnp
import numpy as np
from jax import lax
from jax.experimental import pallas as pl
from jax.experimental.pallas import tpu as pltpu
from jax.experimental.pallas import tpu_sc as plsc

_N, _E, _D, _DE, _G, _DB, _T = 10000, 320000, 128, 16, 128, 256, 5
_NTILES = 32
_EPAD = 327680
_EPT = _EPAD // _NTILES
_NPAD = 10240
_RPT = _NPAD // 16


def _edge_mlp_body(ea_ref, we_ref, be_ref, out_ref):
    out_ref[...] = (
        jnp.dot(ea_ref[...], we_ref[...], preferred_element_type=jnp.float32)
        + be_ref[...]
    ).astype(jnp.bfloat16)


def _edge_mlp(ea_p, We, be_row):
    blk = 4096
    return pl.pallas_call(
        _edge_mlp_body,
        grid=(_EPAD // blk,),
        in_specs=[
            pl.BlockSpec((blk, _DE), lambda i: (i, 0)),
            pl.BlockSpec((_DE, _D), lambda i: (0, 0)),
            pl.BlockSpec((1, _D), lambda i: (0, 0)),
        ],
        out_specs=pl.BlockSpec((blk, _D), lambda i: (i, 0)),
        out_shape=jax.ShapeDtypeStruct((_EPAD, _D), jnp.bfloat16),
    )(ea_p, We, be_row)


_CH = 64
_DW = _D // 2


def _edge_agg(src_p, dst_p, x, m):
    mesh = plsc.VectorSubcoreMesh(core_axis_name="c", subcore_axis_name="s")
    nch = _EPT // _CH
    half = nch // 2

    @functools.partial(
        pl.kernel,
        mesh=mesh,
        out_type=jax.ShapeDtypeStruct((2, _NPAD, _D), jnp.float32),
        scratch_types=[
            pltpu.VMEM((_CH,), jnp.int32),
            pltpu.VMEM((_CH,), jnp.int32),
            pltpu.VMEM((_CH,), jnp.int32),
            pltpu.VMEM((_CH,), jnp.int32),
            pltpu.VMEM((_CH, _D), jnp.float32),
            pltpu.VMEM((_CH, _DW), jnp.int32),
            pltpu.VMEM((_CH, _D), jnp.float32),
            pltpu.VMEM((_CH, _DW), jnp.int32),
            pltpu.VMEM((_CH, _D), jnp.float32),
            pltpu.VMEM_SHARED((_NPAD, _D), jnp.float32),
            pltpu.SemaphoreType.DMA,
            pltpu.SemaphoreType.DMA,
            pltpu.SemaphoreType.DMA,
            pltpu.SemaphoreType.DMA,
            pltpu.SemaphoreType.DMA,
            pltpu.SemaphoreType.DMA,
        ],
    )
    def k(src_hbm, dst_hbm, x_hbm, m_hbm, out_hbm, sv0, dv0, sv1, dv1,
          xr0, mr0, xr1, mr1, sr, agg_sh, i0, i1, g0, g1, m0, m1):
        cid = lax.axis_index("c")
        sid = lax.axis_index("s")
        wid = cid * 16 + sid
        ebase = wid * _EPT

        zero = jnp.zeros((16,), jnp.float32)

        def zrow(r, _):
            for g in range(8):
                xr0[r, pl.ds(g * 16, 16)] = zero
            return 0

        lax.fori_loop(0, _CH, zrow, 0)

        def zcp(j, _):
            pltpu.sync_copy(xr0,
                            agg_sh.at[pl.ds(sid * _RPT + j * _CH, _CH), :])
            return 0

        lax.fori_loop(0, _RPT // _CH, zcp, 0)
        plsc.subcore_barrier()

        def issue_idx(j, sv, dv, isem):
            pltpu.async_copy(src_hbm.at[pl.ds(ebase + j * _CH, _CH)], sv, isem)
            pltpu.async_copy(dst_hbm.at[pl.ds(ebase + j * _CH, _CH)], dv, isem)

        def wait_idx(j, sv, dv, isem):
            pltpu.make_async_copy(
                src_hbm.at[pl.ds(ebase + j * _CH, _CH)], sv, isem).wait()
            pltpu.make_async_copy(
                dst_hbm.at[pl.ds(ebase + j * _CH, _CH)], dv, isem).wait()

        def issue_data(j, sv, xr, mr, gs, ms):
            pltpu.async_copy(x_hbm.at[sv], xr, gs)
            pltpu.async_copy(m_hbm.at[pl.ds(ebase + j * _CH, _CH), :], mr, ms)

        def wait_data(j, sv, xr, mr, gs, ms):
            pltpu.make_async_copy(x_hbm.at[sv], xr, gs).wait()
            pltpu.make_async_copy(
                m_hbm.at[pl.ds(ebase + j * _CH, _CH), :], mr, ms).wait()

        maskv = jnp.full((16,), -65536, jnp.int32)
        sh16 = jnp.full((16,), 16, jnp.int32)

        def process(dv, xr, mr):
            def row(r, _2):
                for rr in range(2):
                    rd = 2 * r + rr
                    for g in range(4):
                        xi = mr[rd, pl.ds(g * 16, 16)]
                        lo = lax.bitcast_convert_type(
                            lax.shift_left(xi, sh16), jnp.float32)
                        hi = lax.bitcast_convert_type(xi & maskv, jnp.float32)
                        sa = pl.ds(32 * g, 16)
                        sb = pl.ds(32 * g + 16, 16)
                        sr[rd, sa] = jnp.maximum(xr[rd, sa] + lo, 0.0)
                        sr[rd, sb] = jnp.maximum(xr[rd, sb] + hi, 0.0)
                return 0

            lax.fori_loop(0, _CH // 2, row, 0)
            pltpu.sync_copy(sr, agg_sh.at[dv], add=True)

        issue_idx(0, sv0, dv0, i0)
        wait_idx(0, sv0, dv0, i0)
        issue_data(0, sv0, xr0, mr0, g0, m0)
        issue_idx(1, sv1, dv1, i1)

        def body(t, _):
            j0 = 2 * t
            wait_idx(j0 + 1, sv1, dv1, i1)
            issue_data(j0 + 1, sv1, xr1, mr1, g1, m1)
            wait_data(j0, sv0, xr0, mr0, g0, m0)
            process(dv0, xr0, mr0)

            @pl.when(t < half - 1)
            def _():
                issue_idx(j0 + 2, sv0, dv0, i0)

            wait_data(j0 + 1, sv1, xr1, mr1, g1, m1)
            process(dv1, xr1, mr1)

            @pl.when(t < half - 1)
            def _():
                wait_idx(j0 + 2, sv0, dv0, i0)
                issue_data(j0 + 2, sv0, xr0, mr0, g0, m0)
                issue_idx(j0 + 3, sv1, dv1, i1)

            return 0

        lax.fori_loop(0, half, body, 0)
        plsc.subcore_barrier()

        def wb(j, _):
            off = sid * _RPT + j * _CH
            pltpu.sync_copy(agg_sh.at[pl.ds(off, _CH), :], xr0)
            pltpu.sync_copy(xr0, out_hbm.at[cid, pl.ds(off, _CH), :])
            return 0

        lax.fori_loop(0, _RPT // _CH, wb, 0)

    return k(src_p, dst_p, x, m)


def _head_body(x_ref, a0_ref, a1_ref, b_ref, w1_ref, b1_ref, w2_ref, b2_ref,
               wb1_ref, bb1_ref, wb2_ref, bb2_ref, wh_ref, bh_ref, out_ref,
               sums_ref, cnt_ref):
    i = pl.program_id(0)

    @pl.when(i == 0)
    def _():
        sums_ref[...] = jnp.zeros_like(sums_ref)
        cnt_ref[...] = jnp.zeros_like(cnt_ref)

    rows = x_ref[...] + a0_ref[...] + a1_ref[...]
    h = jnp.maximum(
        jnp.dot(rows, w1_ref[...], preferred_element_type=jnp.float32)
        + b1_ref[...], 0.0)
    hn = (jnp.dot(h, w2_ref[...], preferred_element_type=jnp.float32)
          + b2_ref[...])
    bblk = b_ref[0]
    p = (lax.broadcasted_iota(jnp.int32, (_G, bblk.shape[1]), 0)
         == bblk).astype(jnp.float32)
    sums_ref[...] += jnp.dot(p, hn, preferred_element_type=jnp.float32)
    cnt_ref[...] += jnp.dot(p, jnp.ones((bblk.shape[1], _D), jnp.float32),
                            preferred_element_type=jnp.float32)

    @pl.when(i == pl.num_programs(0) - 1)
    def _():
        hg = sums_ref[...] / jnp.maximum(cnt_ref[...], 1.0)
        hs = jnp.maximum(
            jnp.dot(hg, wb1_ref[...], preferred_element_type=jnp.float32)
            + bb1_ref[...], 0.0)
        hs2 = jnp.maximum(
            jnp.dot(hs, wb2_ref[...], preferred_element_type=jnp.float32)
            + bb2_ref[...], 0.0)
        out_ref[...] = (
            jnp.dot(hs2, wh_ref[...], preferred_element_type=jnp.float32)
            + bh_ref[...])


def _head(x, a0, a1, batch3, W1, b1r, W2, b2r, Wb1, bb1r, Wb2, bb2r, Whp, bhp):
    nblk, rows = 10, _N // 10
    return pl.pallas_call(
        _head_body,
        grid=(nblk,),
        in_specs=[
            pl.BlockSpec((rows, _D), lambda i: (i, 0)),
            pl.BlockSpec((rows, _D), lambda i: (i, 0)),
            pl.BlockSpec((rows, _D), lambda i: (i, 0)),
            pl.BlockSpec((1, 1, rows), lambda i: (i, 0, 0)),
            pl.BlockSpec((_D, _D), lambda i: (0, 0)),
            pl.BlockSpec((1, _D), lambda i: (0, 0)),
            pl.BlockSpec((_D, _D), lambda i: (0, 0)),
            pl.BlockSpec((1, _D), lambda i: (0, 0)),
            pl.BlockSpec((_D, _D), lambda i: (0, 0)),
            pl.BlockSpec((1, _D), lambda i: (0, 0)),
            pl.BlockSpec((_D, _DB), lambda i: (0, 0)),
            pl.BlockSpec((1, _DB), lambda i: (0, 0)),
            pl.BlockSpec((_DB, _D), lambda i: (0, 0)),
            pl.BlockSpec((1, _D), lambda i: (0, 0)),
        ],
        out_specs=pl.BlockSpec((_G, _D), lambda i: (0, 0)),
        out_shape=jax.ShapeDtypeStruct((_G, _D), jnp.float32),
        scratch_shapes=[
            pltpu.VMEM((_G, _D), jnp.float32),
            pltpu.VMEM((_G, _D), jnp.float32),
        ],
    )(x, a0, a1, batch3, W1, b1r, W2, b2r, Wb1, bb1r, Wb2, bb2r, Whp, bhp)


def kernel(x, edge_index, edge_attr, batch, We, be, W1, b1, W2, b2, Wb1, bb1,
           Wb2, bb2, Wh, bh):
    pad_e = _EPAD - _E
    src_p = jnp.concatenate([edge_index[0],
                             jnp.zeros((pad_e,), jnp.int32)])
    dst_p = jnp.concatenate([edge_index[1],
                             _N + (jnp.arange(pad_e, dtype=jnp.int32)
                                   % (_NPAD - _N))])
    ea_p = jnp.concatenate([edge_attr,
                            jnp.zeros((pad_e, _DE), jnp.float32)])

    m = _edge_mlp(ea_p, We, be.reshape(1, _D))
    m_i32 = lax.bitcast_convert_type(m.reshape(_EPAD, _DW, 2), jnp.int32)

    w = np.arange(_D)
    g, j = w // 32, w % 32
    perm = np.where(j < 16, 32 * g + 2 * (j % 16), 32 * g + 2 * (j % 16) + 1)
    inv = np.argsort(perm)

    agg2 = _edge_agg(src_p, dst_p, x[:, perm], m_i32)

    whp = jnp.pad(Wh, ((0, 0), (0, _D - _T)))
    bhp = jnp.pad(bh, (0, _D - _T)).reshape(1, _D)
    a0 = jnp.take(agg2[0, :_N], jnp.asarray(inv), axis=1)
    a1 = jnp.take(agg2[1, :_N], jnp.asarray(inv), axis=1)
    out = _head(x, a0, a1, batch.reshape(10, 1, _N // 10),
                W1, b1.reshape(1, _D), W2, b2.reshape(1, _D),
                Wb1, bb1.reshape(1, _D), Wb2, bb2.reshape(1, _DB), whp, bhp)
    return out[:, :_T]


# speedup vs baseline: 2.0357x; 2.0357x over previous
import functools

import jax
import jax.numpy as jnp
import numpy as np
from jax import lax
from jax.experimental import pallas as pl
from jax.experimental.pallas import tpu as pltpu
from jax.experimental.pallas import tpu_sc as plsc

_N, _E, _D, _DE, _G, _DB, _T = 10000, 320000, 128, 16, 128, 256, 5
_NTILES = 32
_EPAD = 327680
_EPT = _EPAD // _NTILES
_NPAD = 10240
_RPT = _NPAD // 16


def _edge_mlp_body(ea_ref, we_ref, be_ref, out_ref):
    out_ref[...] = (
        jnp.dot(ea_ref[...], we_ref[...], preferred_element_type=jnp.float32)
        + be_ref[...]
    )


def _edge_mlp(ea_p, We, be_row):
    blk = 4096
    return pl.pallas_call(
        _edge_mlp_body,
        grid=(_EPAD // blk,),
        in_specs=[
            pl.BlockSpec((blk, _DE), lambda i: (i, 0)),
            pl.BlockSpec((_DE, _D), lambda i: (0, 0)),
            pl.BlockSpec((1, _D), lambda i: (0, 0)),
        ],
        out_specs=pl.BlockSpec((blk, _D), lambda i: (i, 0)),
        out_shape=jax.ShapeDtypeStruct((_EPAD, _D), jnp.float32),
    )(ea_p, We, be_row)


_CH = 80


def _edge_agg(src_p, dst_p, x, m):
    mesh = plsc.VectorSubcoreMesh(core_axis_name="c", subcore_axis_name="s")
    nch = _EPT // _CH
    half = nch // 2

    @functools.partial(
        pl.kernel,
        mesh=mesh,
        out_type=jax.ShapeDtypeStruct((2, _NPAD, _D), jnp.float32),
        scratch_types=[
            pltpu.VMEM((_CH,), jnp.int32),
            pltpu.VMEM((_CH,), jnp.int32),
            pltpu.VMEM((_CH,), jnp.int32),
            pltpu.VMEM((_CH,), jnp.int32),
            pltpu.VMEM((_CH, _D), jnp.float32),
            pltpu.VMEM((_CH, _D), jnp.float32),
            pltpu.VMEM((_CH, _D), jnp.float32),
            pltpu.VMEM((_CH, _D), jnp.float32),
            pltpu.VMEM_SHARED((_NPAD, _D), jnp.float32),
            pltpu.SemaphoreType.DMA,
            pltpu.SemaphoreType.DMA,
            pltpu.SemaphoreType.DMA,
            pltpu.SemaphoreType.DMA,
            pltpu.SemaphoreType.DMA,
            pltpu.SemaphoreType.DMA,
        ],
    )
    def k(src_hbm, dst_hbm, x_hbm, m_hbm, out_hbm, sv0, dv0, sv1, dv1,
          xr0, mr0, xr1, mr1, agg_sh, i0, i1, g0, g1, m0, m1):
        cid = lax.axis_index("c")
        sid = lax.axis_index("s")
        wid = cid * 16 + sid
        ebase = wid * _EPT

        zero = jnp.zeros((16,), jnp.float32)

        def zrow(r, _):
            for g in range(8):
                xr0[r, pl.ds(g * 16, 16)] = zero
            return 0

        lax.fori_loop(0, _CH, zrow, 0)

        def zcp(j, _):
            pltpu.sync_copy(xr0,
                            agg_sh.at[pl.ds(sid * _RPT + j * _CH, _CH), :])
            return 0

        lax.fori_loop(0, _RPT // _CH, zcp, 0)
        plsc.subcore_barrier()

        def issue_idx(j, sv, dv, isem):
            pltpu.async_copy(src_hbm.at[pl.ds(ebase + j * _CH, _CH)], sv, isem)
            pltpu.async_copy(dst_hbm.at[pl.ds(ebase + j * _CH, _CH)], dv, isem)

        def wait_idx(j, sv, dv, isem):
            pltpu.make_async_copy(
                src_hbm.at[pl.ds(ebase + j * _CH, _CH)], sv, isem).wait()
            pltpu.make_async_copy(
                dst_hbm.at[pl.ds(ebase + j * _CH, _CH)], dv, isem).wait()

        def issue_data(j, sv, xr, mr, gs, ms):
            pltpu.async_copy(x_hbm.at[sv], xr, gs)
            pltpu.async_copy(m_hbm.at[pl.ds(ebase + j * _CH, _CH), :], mr, ms)

        def wait_data(j, sv, xr, mr, gs, ms):
            pltpu.make_async_copy(x_hbm.at[sv], xr, gs).wait()
            pltpu.make_async_copy(
                m_hbm.at[pl.ds(ebase + j * _CH, _CH), :], mr, ms).wait()

        def process(dv, xr, mr):
            def row(r, _2):
                for rr in range(2):
                    for g in range(8):
                        sl = pl.ds(g * 16, 16)
                        mr[2 * r + rr, sl] = jnp.maximum(
                            xr[2 * r + rr, sl] + mr[2 * r + rr, sl], 0.0)
                return 0

            lax.fori_loop(0, _CH // 2, row, 0)
            pltpu.sync_copy(mr, agg_sh.at[dv], add=True)

        issue_idx(0, sv0, dv0, i0)
        wait_idx(0, sv0, dv0, i0)
        issue_data(0, sv0, xr0, mr0, g0, m0)
        issue_idx(1, sv1, dv1, i1)

        def body(t, _):
            j0 = 2 * t
            wait_idx(j0 + 1, sv1, dv1, i1)
            issue_data(j0 + 1, sv1, xr1, mr1, g1, m1)
            wait_data(j0, sv0, xr0, mr0, g0, m0)
            process(dv0, xr0, mr0)

            @pl.when(t < half - 1)
            def _():
                issue_idx(j0 + 2, sv0, dv0, i0)

            wait_data(j0 + 1, sv1, xr1, mr1, g1, m1)
            process(dv1, xr1, mr1)

            @pl.when(t < half - 1)
            def _():
                wait_idx(j0 + 2, sv0, dv0, i0)
                issue_data(j0 + 2, sv0, xr0, mr0, g0, m0)
                issue_idx(j0 + 3, sv1, dv1, i1)

            return 0

        lax.fori_loop(0, half, body, 0)
        plsc.subcore_barrier()

        def wb(j, _):
            off = sid * _RPT + j * _CH
            pltpu.sync_copy(agg_sh.at[pl.ds(off, _CH), :], xr0)
            pltpu.sync_copy(xr0, out_hbm.at[cid, pl.ds(off, _CH), :])
            return 0

        lax.fori_loop(0, _RPT // _CH, wb, 0)

    return k(src_p, dst_p, x, m)


def _head_body(x_ref, a0_ref, a1_ref, b_ref, w1_ref, b1_ref, w2_ref, b2_ref,
               wb1_ref, bb1_ref, wb2_ref, bb2_ref, wh_ref, bh_ref, out_ref,
               sums_ref, cnt_ref):
    i = pl.program_id(0)

    @pl.when(i == 0)
    def _():
        sums_ref[...] = jnp.zeros_like(sums_ref)
        cnt_ref[...] = jnp.zeros_like(cnt_ref)

    rows = x_ref[...] + a0_ref[...] + a1_ref[...]
    h = jnp.maximum(
        jnp.dot(rows, w1_ref[...], preferred_element_type=jnp.float32)
        + b1_ref[...], 0.0)
    hn = (jnp.dot(h, w2_ref[...], preferred_element_type=jnp.float32)
          + b2_ref[...])
    bblk = b_ref[0]
    p = (lax.broadcasted_iota(jnp.int32, (_G, bblk.shape[1]), 0)
         == bblk).astype(jnp.float32)
    sums_ref[...] += jnp.dot(p, hn, preferred_element_type=jnp.float32)
    cnt_ref[...] += jnp.dot(p, jnp.ones((bblk.shape[1], _D), jnp.float32),
                            preferred_element_type=jnp.float32)

    @pl.when(i == pl.num_programs(0) - 1)
    def _():
        hg = sums_ref[...] / jnp.maximum(cnt_ref[...], 1.0)
        hs = jnp.maximum(
            jnp.dot(hg, wb1_ref[...], preferred_element_type=jnp.float32)
            + bb1_ref[...], 0.0)
        hs2 = jnp.maximum(
            jnp.dot(hs, wb2_ref[...], preferred_element_type=jnp.float32)
            + bb2_ref[...], 0.0)
        out_ref[...] = (
            jnp.dot(hs2, wh_ref[...], preferred_element_type=jnp.float32)
            + bh_ref[...])


def _head(x, a0, a1, batch3, W1, b1r, W2, b2r, Wb1, bb1r, Wb2, bb2r, Whp, bhp):
    nblk, rows = 10, _N // 10
    return pl.pallas_call(
        _head_body,
        grid=(nblk,),
        in_specs=[
            pl.BlockSpec((rows, _D), lambda i: (i, 0)),
            pl.BlockSpec((rows, _D), lambda i: (i, 0)),
            pl.BlockSpec((rows, _D), lambda i: (i, 0)),
            pl.BlockSpec((1, 1, rows), lambda i: (i, 0, 0)),
            pl.BlockSpec((_D, _D), lambda i: (0, 0)),
            pl.BlockSpec((1, _D), lambda i: (0, 0)),
            pl.BlockSpec((_D, _D), lambda i: (0, 0)),
            pl.BlockSpec((1, _D), lambda i: (0, 0)),
            pl.BlockSpec((_D, _D), lambda i: (0, 0)),
            pl.BlockSpec((1, _D), lambda i: (0, 0)),
            pl.BlockSpec((_D, _DB), lambda i: (0, 0)),
            pl.BlockSpec((1, _DB), lambda i: (0, 0)),
            pl.BlockSpec((_DB, _D), lambda i: (0, 0)),
            pl.BlockSpec((1, _D), lambda i: (0, 0)),
        ],
        out_specs=pl.BlockSpec((_G, _D), lambda i: (0, 0)),
        out_shape=jax.ShapeDtypeStruct((_G, _D), jnp.float32),
        scratch_shapes=[
            pltpu.VMEM((_G, _D), jnp.float32),
            pltpu.VMEM((_G, _D), jnp.float32),
        ],
    )(x, a0, a1, batch3, W1, b1r, W2, b2r, Wb1, bb1r, Wb2, bb2r, Whp, bhp)


def kernel(x, edge_index, edge_attr, batch, We, be, W1, b1, W2, b2, Wb1, bb1,
           Wb2, bb2, Wh, bh):
    pad_e = _EPAD - _E
    src_p = jnp.concatenate([edge_index[0],
                             jnp.zeros((pad_e,), jnp.int32)])
    dst_p = jnp.concatenate([edge_index[1],
                             _N + (jnp.arange(pad_e, dtype=jnp.int32)
                                   % (_NPAD - _N))])
    ea_p = jnp.concatenate([edge_attr,
                            jnp.zeros((pad_e, _DE), jnp.float32)])

    m = _edge_mlp(ea_p, We, be.reshape(1, _D))
    agg2 = _edge_agg(src_p, dst_p, x, m)

    whp = jnp.pad(Wh, ((0, 0), (0, _D - _T)))
    bhp = jnp.pad(bh, (0, _D - _T)).reshape(1, _D)
    out = _head(x, agg2[0, :_N], agg2[1, :_N], batch.reshape(10, 1, _N // 10),
                W1, b1.reshape(1, _D), W2, b2.reshape(1, _D),
                Wb1, bb1.reshape(1, _D), Wb2, bb2.reshape(1, _DB), whp, bhp)
    return out[:, :_T]
